# baseline (device time: 63010 ns/iter reference)
import jax
import jax.numpy as jnp
from jax import lax
from jax.experimental import pallas as pl
from jax.experimental.pallas import tpu as pltpu

_CHUNKS_PER_B = 4


def kernel(O, Wo):
    B, S, H, D = O.shape
    K = H * D
    N = Wo.shape[1]
    Nh = N // 2
    S_half = S // 2
    C = _CHUNKS_PER_B
    rows = S_half // C
    n_chunks = B * C

    Ot = jnp.transpose(O.reshape(B, S, K), (0, 2, 1))

    dn = (((0,), (0,)), ((), ()))

    def body(ot_hbm, w_hbm, out_hbm, stage, w_stage, send_buf, recv_buf,
             acc_buf, in_sems, w_sems, send_sems, recv_sems, copy_sems):
        my_x = lax.axis_index("x")
        my_y = lax.axis_index("y")
        my_z = lax.axis_index("z")
        peer = (1 - my_x, my_y, my_z)

        my_lo = my_x * S_half
        peer_lo = (1 - my_x) * S_half

        for h in range(2):
            pltpu.make_async_copy(
                w_hbm.at[:, h * Nh:(h + 1) * Nh],
                w_stage.at[:, h * Nh:(h + 1) * Nh],
                w_sems.at[h],
            ).start()

        def o_chunk_copy(i, b, lo, c):
            return pltpu.make_async_copy(
                ot_hbm.at[b, :, pl.ds(lo + c * rows, rows)],
                stage.at[b, :, pl.ds(lo + c * rows, rows)],
                in_sems.at[i],
            )

        for b in range(B):
            for c in range(C):
                o_chunk_copy(b * C + c, b, peer_lo, c).start()
        for b in range(B):
            pltpu.make_async_copy(
                ot_hbm.at[b, :, pl.ds(my_lo, S_half)],
                stage.at[b, :, pl.ds(my_lo, S_half)],
                in_sems.at[n_chunks + b],
            ).start()

        barrier = pltpu.get_barrier_semaphore()
        pl.semaphore_signal(
            barrier, inc=1, device_id=peer, device_id_type=pl.DeviceIdType.MESH
        )
        pl.semaphore_wait(barrier, 1)

        pltpu.make_async_copy(w_hbm.at[:, :Nh], w_stage.at[:, :Nh],
                              w_sems.at[0]).wait()
        w0 = w_stage[:, :Nh].astype(jnp.bfloat16)
        pltpu.make_async_copy(w_hbm.at[:, Nh:], w_stage.at[:, Nh:],
                              w_sems.at[1]).wait()
        w1 = w_stage[:, Nh:].astype(jnp.bfloat16)

        def partial_chunk(dst, b, lo, c):
            a_t = stage[b, :, pl.ds(lo + c * rows, rows)].astype(jnp.bfloat16)
            sl = pl.ds(c * rows, rows)
            dst[b, sl, :Nh] = lax.dot_general(
                a_t, w0, dn, preferred_element_type=jnp.float32
            ).astype(dst.dtype)
            dst[b, sl, Nh:] = lax.dot_general(
                a_t, w1, dn, preferred_element_type=jnp.float32
            ).astype(dst.dtype)

        def chunk_rdma(b, c):
            i = b * C + c
            return pltpu.make_async_remote_copy(
                src_ref=send_buf.at[b, c * rows:(c + 1) * rows],
                dst_ref=recv_buf.at[b, c * rows:(c + 1) * rows],
                send_sem=send_sems.at[i],
                recv_sem=recv_sems.at[i],
                device_id=peer,
                device_id_type=pl.DeviceIdType.MESH,
            )

        for b in range(B):
            for c in range(C):
                o_chunk_copy(b * C + c, b, peer_lo, c).wait()
                partial_chunk(send_buf, b, peer_lo, c)
                chunk_rdma(b, c).start()

        for b in range(B):
            pltpu.make_async_copy(
                ot_hbm.at[b, :, pl.ds(my_lo, S_half)],
                stage.at[b, :, pl.ds(my_lo, S_half)],
                in_sems.at[n_chunks + b],
            ).wait()
            for c in range(C):
                partial_chunk(acc_buf, b, my_lo, c)

        for b in range(B):
            for c in range(C):
                chunk_rdma(b, c).wait_recv()
                sl = pl.ds(c * rows, rows)
                acc_buf[b, sl] = acc_buf[b, sl] + recv_buf[
                    b, c * rows:(c + 1) * rows
                ].astype(jnp.float32)
                pltpu.make_async_copy(
                    acc_buf.at[b, c * rows:(c + 1) * rows],
                    out_hbm.at[b, c * rows:(c + 1) * rows],
                    copy_sems.at[b * C + c],
                ).start()

        for b in range(B):
            for c in range(C):
                pltpu.make_async_copy(
                    acc_buf.at[b, c * rows:(c + 1) * rows],
                    out_hbm.at[b, c * rows:(c + 1) * rows],
                    copy_sems.at[b * C + c],
                ).wait()
                chunk_rdma(b, c).wait_send()

    return pl.pallas_call(
        body,
        out_shape=jax.ShapeDtypeStruct((B, S_half, N), jnp.float32),
        in_specs=[
            pl.BlockSpec(memory_space=pl.ANY),
            pl.BlockSpec(memory_space=pl.ANY),
        ],
        out_specs=pl.BlockSpec(memory_space=pltpu.MemorySpace.HBM),
        scratch_shapes=[
            pltpu.VMEM((B, K, S), jnp.float32),
            pltpu.VMEM((K, N), jnp.float32),
            pltpu.VMEM((B, S_half, N), jnp.bfloat16),
            pltpu.VMEM((B, S_half, N), jnp.bfloat16),
            pltpu.VMEM((B, S_half, N), jnp.float32),
            pltpu.SemaphoreType.DMA((n_chunks + B,)),
            pltpu.SemaphoreType.DMA((2,)),
            pltpu.SemaphoreType.DMA((n_chunks,)),
            pltpu.SemaphoreType.DMA((n_chunks,)),
            pltpu.SemaphoreType.DMA((n_chunks,)),
        ],
        compiler_params=pltpu.CompilerParams(
            collective_id=0, vmem_limit_bytes=100 * 1024 * 1024
        ),
    )(Ot, Wo)


# device time: 62654 ns/iter; 1.0057x vs baseline; 1.0057x over previous
import jax
import jax.numpy as jnp
from jax import lax
from jax.experimental import pallas as pl
from jax.experimental.pallas import tpu as pltpu

_CHUNKS_PER_B = 4


def kernel(O, Wo):
    B, S, H, D = O.shape
    K = H * D
    N = Wo.shape[1]
    Nh = N // 2
    S_half = S // 2
    C = _CHUNKS_PER_B
    rows = S_half // C
    n_send = B * C * 2

    Ot = jnp.transpose(O.reshape(B, S, K), (0, 2, 1))

    dn = (((0,), (0,)), ((), ()))

    def body(ot_hbm, w_hbm, out_ref, stage, w_stage, send_buf, recv_buf,
             in_sems, w_sems, send_sems, recv_sems):
        my_x = lax.axis_index("x")
        my_y = lax.axis_index("y")
        my_z = lax.axis_index("z")
        peer = (1 - my_x, my_y, my_z)

        my_lo = my_x * S_half
        peer_lo = (1 - my_x) * S_half

        def w_copy(h):
            return pltpu.make_async_copy(
                w_hbm.at[:, h * Nh:(h + 1) * Nh],
                w_stage.at[:, h * Nh:(h + 1) * Nh],
                w_sems.at[h],
            )

        def o_copy(b):
            return pltpu.make_async_copy(
                ot_hbm.at[b], stage.at[b], in_sems.at[b]
            )

        w_copy(0).start()
        o_copy(0).start()
        w_copy(1).start()
        o_copy(1).start()

        barrier = pltpu.get_barrier_semaphore()
        pl.semaphore_signal(
            barrier, inc=1, device_id=peer, device_id_type=pl.DeviceIdType.MESH
        )
        pl.semaphore_wait(barrier, 1)

        w_copy(0).wait()
        w0 = w_stage[:, :Nh].astype(jnp.bfloat16)

        def dot_piece(b, lo, c, h, w_val):
            a_t = stage[b, :, pl.ds(lo + c * rows, rows)].astype(jnp.bfloat16)
            return lax.dot_general(
                a_t, w_val, dn, preferred_element_type=jnp.float32
            )

        def piece_rdma(b, c, h):
            i = (b * C + c) * 2 + h
            rs = slice(c * rows, (c + 1) * rows)
            cs = slice(h * Nh, (h + 1) * Nh)
            return pltpu.make_async_remote_copy(
                src_ref=send_buf.at[b, rs, cs],
                dst_ref=recv_buf.at[b, rs, cs],
                send_sem=send_sems.at[i],
                recv_sem=recv_sems.at[i],
                device_id=peer,
                device_id_type=pl.DeviceIdType.MESH,
            )

        w1 = None
        for b in range(B):
            o_copy(b).wait()
            for h in range(2):
                if h == 1 and w1 is None:
                    w_copy(1).wait()
                    w1 = w_stage[:, Nh:].astype(jnp.bfloat16)
                w_val = w0 if h == 0 else w1
                for c in range(C):
                    send_buf[b, c * rows:(c + 1) * rows,
                             h * Nh:(h + 1) * Nh] = dot_piece(
                        b, peer_lo, c, h, w_val
                    ).astype(jnp.bfloat16)
                    piece_rdma(b, c, h).start()

        for b in range(B):
            for h in range(2):
                w_val = w0 if h == 0 else w1
                for c in range(C):
                    out_ref[b, c * rows:(c + 1) * rows,
                            h * Nh:(h + 1) * Nh] = dot_piece(
                        b, my_lo, c, h, w_val
                    )

        for b in range(B):
            for h in range(2):
                for c in range(C):
                    piece_rdma(b, c, h).wait_recv()
                    rs = slice(c * rows, (c + 1) * rows)
                    cs = slice(h * Nh, (h + 1) * Nh)
                    out_ref[b, rs, cs] = out_ref[b, rs, cs] + recv_buf[
                        b, rs, cs
                    ].astype(jnp.float32)

        for b in range(B):
            for h in range(2):
                for c in range(C):
                    piece_rdma(b, c, h).wait_send()

    return pl.pallas_call(
        body,
        out_shape=jax.ShapeDtypeStruct((B, S_half, N), jnp.float32),
        in_specs=[
            pl.BlockSpec(memory_space=pl.ANY),
            pl.BlockSpec(memory_space=pl.ANY),
        ],
        out_specs=pl.BlockSpec(memory_space=pltpu.VMEM),
        scratch_shapes=[
            pltpu.VMEM((B, K, S), jnp.float32),
            pltpu.VMEM((K, N), jnp.float32),
            pltpu.VMEM((B, S_half, N), jnp.bfloat16),
            pltpu.VMEM((B, S_half, N), jnp.bfloat16),
            pltpu.SemaphoreType.DMA((B,)),
            pltpu.SemaphoreType.DMA((2,)),
            pltpu.SemaphoreType.DMA((n_send,)),
            pltpu.SemaphoreType.DMA((n_send,)),
        ],
        compiler_params=pltpu.CompilerParams(
            collective_id=0, vmem_limit_bytes=100 * 1024 * 1024
        ),
    )(Ot, Wo)
